# baseline (speedup 1.0000x reference)
import functools

import jax
import jax.numpy as jnp
from jax import lax
from jax.experimental import pallas as pl
from jax.experimental.pallas import tpu as pltpu
from jax.experimental.pallas import tpu_sc as plsc

N = 10000
E = 320000
B = 16
HID = 64
NPAD = 10240
EPAD = 327680
NTILES = 32
EPW = EPAD // NTILES
CH = 128
NCHUNK = EPW // CH
RPT = NPAD // 16
DEGW = 16
NC0 = 58
NC1 = 102
NCMAX = max(NC0, NC1)
BLK = 1280
NBLK = NPAD // BLK


@functools.cache
def _mesh():
    return plsc.VectorSubcoreMesh(
        core_axis_name="c", subcore_axis_name="s",
        num_cores=2, num_subcores=16)


def _fill(ref, rows, width, value):
    vec = jnp.full((16,), value, jnp.float32)

    def body(i, carry):
        for j in range(width // 16):
            ref[i, pl.ds(j * 16, 16)] = vec
        return carry

    lax.fori_loop(0, rows, body, 0)


def _sc_scatter_body(y_hbm, src_hbm, dst_hbm, out_hbm,
                     sidx, didx, rows0, rows1, zbuf, acc, sem0, sem1):
    cid = lax.axis_index("c")
    sid = lax.axis_index("s")
    _fill(zbuf, CH, HID, 0.0)
    row0 = sid * RPT
    for k in range(RPT // CH):
        pltpu.sync_copy(zbuf, acc.at[pl.ds(row0 + k * CH, CH)])
    @pl.when(cid == 0)
    def _stage0():
        c0 = sid * (NC0 + NC1)
        pltpu.sync_copy(src_hbm.at[pl.ds(c0, NC0)], sidx.at[pl.ds(0, NC0)])
        pltpu.sync_copy(dst_hbm.at[pl.ds(c0, NC0)], didx.at[pl.ds(0, NC0)])

    @pl.when(cid == 1)
    def _stage1():
        c0 = sid * (NC0 + NC1) + NC0
        pltpu.sync_copy(src_hbm.at[pl.ds(c0, NC1)], sidx.at[pl.ds(0, NC1)])
        pltpu.sync_copy(dst_hbm.at[pl.ds(c0, NC1)], didx.at[pl.ds(0, NC1)])

    plsc.subcore_barrier()

    def body(c, carry):
        pltpu.sync_copy(y_hbm.at[sidx.at[c]], rows0)
        pltpu.sync_copy(rows0, acc.at[didx.at[c]], add=True)
        return carry

    @pl.when(cid == 0)
    def _loop0():
        lax.fori_loop(0, NC0, body, 0)

    @pl.when(cid == 1)
    def _loop1():
        lax.fori_loop(0, NC1, body, 0)

    plsc.subcore_barrier()
    pltpu.sync_copy(acc.at[pl.ds(row0, RPT)],
                    out_hbm.at[cid, pl.ds(row0, RPT)])


def _sc_scatter(y, srcp, dstp):
    return pl.kernel(
        _sc_scatter_body,
        out_type=jax.ShapeDtypeStruct((2, NPAD, HID), jnp.float32),
        mesh=_mesh(),
        compiler_params=pltpu.CompilerParams(use_tc_tiling_on_sc=False),
        scratch_types=[
            pltpu.VMEM((NCMAX + 1, CH), jnp.int32),
            pltpu.VMEM((NCMAX, CH), jnp.int32),
            pltpu.VMEM((CH, HID), jnp.float32),
            pltpu.VMEM((CH, HID), jnp.float32),
            pltpu.VMEM((CH, HID), jnp.float32),
            pltpu.VMEM_SHARED((NPAD, HID), jnp.float32),
            pltpu.SemaphoreType.DMA,
            pltpu.SemaphoreType.DMA,
        ],
    )(y, srcp, dstp)


def _sc_degree_body(dst_hbm, out_hbm, didx, ones_v, zbuf, acc):
    cid = lax.axis_index("c")
    sid = lax.axis_index("s")
    wid = sid * 2 + cid
    _fill(zbuf, CH, DEGW, 0.0)
    _fill(ones_v, CH, DEGW, 1.0)
    row0 = sid * RPT
    for k in range(RPT // CH):
        pltpu.sync_copy(zbuf, acc.at[pl.ds(row0 + k * CH, CH)])
    pltpu.sync_copy(dst_hbm.at[pl.ds(wid * NCHUNK, NCHUNK)], didx)
    plsc.subcore_barrier()

    def body(c, carry):
        pltpu.sync_copy(ones_v, acc.at[didx.at[c]], add=True)
        return carry

    lax.fori_loop(0, NCHUNK, body, 0)
    plsc.subcore_barrier()
    pltpu.sync_copy(acc.at[pl.ds(row0, RPT)],
                    out_hbm.at[cid, pl.ds(row0, RPT)])


def _sc_degree(dstp):
    return pl.kernel(
        _sc_degree_body,
        out_type=jax.ShapeDtypeStruct((2, NPAD, DEGW), jnp.float32),
        mesh=_mesh(),
        compiler_params=pltpu.CompilerParams(use_tc_tiling_on_sc=False),
        scratch_types=[
            pltpu.VMEM((NCHUNK, CH), jnp.int32),
            pltpu.VMEM((CH, DEGW), jnp.float32),
            pltpu.VMEM((CH, DEGW), jnp.float32),
            pltpu.VMEM_SHARED((NPAD, DEGW), jnp.float32),
        ],
    )(dstp)


def _tc_embed_body(xp, req, ts, b2d, degp, wxp, wea, wet, bemb, w1,
                   y1_o, dinv_o):
    f32 = jnp.float32
    oh = (b2d[...] == lax.broadcasted_iota(jnp.int32, (BLK, B), 1)).astype(f32)
    z = req[:, 0:1] * wea[...] + jnp.dot(ts[...], wet[...],
                                         preferred_element_type=f32,
                  precision=lax.Precision.HIGHEST)
    h0 = (jnp.dot(xp[...], wxp[...], preferred_element_type=f32,
                  precision=lax.Precision.HIGHEST)
          + jnp.dot(oh, z, preferred_element_type=f32,
                  precision=lax.Precision.HIGHEST) + bemb[...])
    deg = degp[0, :, 0:1] + degp[1, :, 0:1] + 1.0
    dinv = 1.0 / jnp.sqrt(deg)
    dinv_o[...] = dinv
    y1_o[...] = dinv * jnp.dot(h0, w1[...], preferred_element_type=f32,
                  precision=lax.Precision.HIGHEST)


def _tc_embed(xp, req, ts, b2d, degp, wxp, wea, wet, bemb, w1):
    return pl.pallas_call(
        _tc_embed_body,
        grid=(NBLK,),
        in_specs=[
            pl.BlockSpec((BLK, 128), lambda i: (i, 0)),
            pl.BlockSpec((B, 4), lambda i: (0, 0)),
            pl.BlockSpec((B, 4), lambda i: (0, 0)),
            pl.BlockSpec((BLK, 1), lambda i: (i, 0)),
            pl.BlockSpec((2, BLK, DEGW), lambda i: (0, i, 0)),
            pl.BlockSpec((128, HID), lambda i: (0, 0)),
            pl.BlockSpec((1, HID), lambda i: (0, 0)),
            pl.BlockSpec((4, HID), lambda i: (0, 0)),
            pl.BlockSpec((1, HID), lambda i: (0, 0)),
            pl.BlockSpec((HID, HID), lambda i: (0, 0)),
        ],
        out_specs=[
            pl.BlockSpec((BLK, HID), lambda i: (i, 0)),
            pl.BlockSpec((BLK, 1), lambda i: (i, 0)),
        ],
        out_shape=[
            jax.ShapeDtypeStruct((NPAD, HID), jnp.float32),
            jax.ShapeDtypeStruct((NPAD, 1), jnp.float32),
        ],
    )(xp, req, ts, b2d, degp, wxp, wea, wet, bemb, w1)


def _pool_update(h, b2d, s_acc, c_acc, m_acc):
    f32 = jnp.float32
    pid = pl.program_id(0)
    oh = (b2d == lax.broadcasted_iota(jnp.int32, (BLK, B), 1)).astype(f32)
    dims = (((0,), (0,)), ((), ()))
    s = lax.dot_general(oh, h, dims, preferred_element_type=f32,
                  precision=lax.Precision.HIGHEST)
    c = lax.dot_general(oh, jnp.ones((BLK, 1), f32), dims,
                        preferred_element_type=f32,
                  precision=lax.Precision.HIGHEST)
    neg = jnp.float32(-jnp.inf)
    mxs = [jnp.max(jnp.where(b2d == g, h, neg), axis=0, keepdims=True)
           for g in range(B)]
    m = jnp.concatenate(mxs, axis=0)

    @pl.when(pid == 0)
    def _init():
        s_acc[...] = s
        c_acc[...] = c
        m_acc[...] = m

    @pl.when(pid > 0)
    def _upd():
        s_acc[...] += s
        c_acc[...] += c
        m_acc[...] = jnp.maximum(m_acc[...], m)


def _layer_h(accp, yprev, dinv, bl):
    return jnp.maximum(
        dinv[...] * (accp[0] + accp[1] + yprev[...]) + bl[...], 0.0)


def _tc_layer_body(accp, yprev, dinv, b2d, bl, wnext,
                   xl_o, ynext_o, s_acc, c_acc, m_acc):
    f32 = jnp.float32
    h = _layer_h(accp, yprev, dinv, bl)
    _pool_update(h, b2d[...], s_acc, c_acc, m_acc)
    ynext_o[...] = dinv[...] * jnp.dot(h, wnext[...],
                                       preferred_element_type=f32,
                  precision=lax.Precision.HIGHEST)

    @pl.when(pl.program_id(0) == NBLK - 1)
    def _fin():
        mean = s_acc[...] / jnp.maximum(c_acc[...], 1.0)
        xl_o[...] = jnp.concatenate([mean, m_acc[...]], axis=1)


def _tc_layer(accp, yprev, dinv, b2d, bl, wnext):
    return pl.pallas_call(
        _tc_layer_body,
        grid=(NBLK,),
        in_specs=[
            pl.BlockSpec((2, BLK, HID), lambda i: (0, i, 0)),
            pl.BlockSpec((BLK, HID), lambda i: (i, 0)),
            pl.BlockSpec((BLK, 1), lambda i: (i, 0)),
            pl.BlockSpec((BLK, 1), lambda i: (i, 0)),
            pl.BlockSpec((1, HID), lambda i: (0, 0)),
            pl.BlockSpec((HID, HID), lambda i: (0, 0)),
        ],
        out_specs=[
            pl.BlockSpec((B, 2 * HID), lambda i: (0, 0)),
            pl.BlockSpec((BLK, HID), lambda i: (i, 0)),
        ],
        out_shape=[
            jax.ShapeDtypeStruct((B, 2 * HID), jnp.float32),
            jax.ShapeDtypeStruct((NPAD, HID), jnp.float32),
        ],
        scratch_shapes=[
            pltpu.VMEM((B, HID), jnp.float32),
            pltpu.VMEM((B, 1), jnp.float32),
            pltpu.VMEM((B, HID), jnp.float32),
        ],
    )(accp, yprev, dinv, b2d, bl, wnext)


def _tc_head_body(accp, yprev, dinv, b2d, bl, x1, x2, wfc1, bfc1, wfc2, bfc2,
                  out_o, s_acc, c_acc, m_acc):
    f32 = jnp.float32
    h = _layer_h(accp, yprev, dinv, bl)
    _pool_update(h, b2d[...], s_acc, c_acc, m_acc)

    @pl.when(pl.program_id(0) == NBLK - 1)
    def _fin():
        mean = s_acc[...] / jnp.maximum(c_acc[...], 1.0)
        x3 = jnp.concatenate([mean, m_acc[...]], axis=1)
        g = x1[...] + x2[...] + x3
        t = jnp.maximum(jnp.dot(g, wfc1[...], preferred_element_type=f32,
                  precision=lax.Precision.HIGHEST)
                        + bfc1[...], 0.0)
        out_o[...] = jnp.dot(t, wfc2[...], preferred_element_type=f32,
                  precision=lax.Precision.HIGHEST) \
            + bfc2[...]


def _tc_head(accp, yprev, dinv, b2d, bl, x1, x2, wfc1, bfc1, wfc2, bfc2):
    return pl.pallas_call(
        _tc_head_body,
        grid=(NBLK,),
        in_specs=[
            pl.BlockSpec((2, BLK, HID), lambda i: (0, i, 0)),
            pl.BlockSpec((BLK, HID), lambda i: (i, 0)),
            pl.BlockSpec((BLK, 1), lambda i: (i, 0)),
            pl.BlockSpec((BLK, 1), lambda i: (i, 0)),
            pl.BlockSpec((1, HID), lambda i: (0, 0)),
            pl.BlockSpec((B, 2 * HID), lambda i: (0, 0)),
            pl.BlockSpec((B, 2 * HID), lambda i: (0, 0)),
            pl.BlockSpec((2 * HID, HID), lambda i: (0, 0)),
            pl.BlockSpec((1, HID), lambda i: (0, 0)),
            pl.BlockSpec((HID, 1), lambda i: (0, 0)),
            pl.BlockSpec((1, 1), lambda i: (0, 0)),
        ],
        out_specs=pl.BlockSpec((B, 1), lambda i: (0, 0)),
        out_shape=jax.ShapeDtypeStruct((B, 1), jnp.float32),
        scratch_shapes=[
            pltpu.VMEM((B, HID), jnp.float32),
            pltpu.VMEM((B, 1), jnp.float32),
            pltpu.VMEM((B, HID), jnp.float32),
        ],
    )(accp, yprev, dinv, b2d, bl, x1, x2, wfc1, bfc1, wfc2, bfc2)


def kernel(x, edge_index, batch, request, timestamp, W_embed, b_embed,
           W1, b1, W2, b2, W3, b3, Wfc1, bfc1, Wfc2, bfc2):
    npad = NPAD - N
    xp = jnp.pad(x, ((0, npad), (0, 5)))
    wxp = jnp.pad(W_embed[:123], ((0, 5), (0, 0)))
    wea = W_embed[123:124]
    wet = W_embed[124:128]
    b2d = jnp.pad(batch[:, None], ((0, npad), (0, 0)),
                  constant_values=B)
    pad = EPAD - E
    srcp = jnp.concatenate(
        [edge_index[0], jnp.zeros((pad,), jnp.int32)]).reshape(
            EPAD // CH, CH)
    dstp = jnp.concatenate(
        [edge_index[1], jnp.full((pad,), N, jnp.int32)]).reshape(
            EPAD // CH, CH)
    bemb = b_embed.reshape(1, HID)
    b1r, b2r, b3r = (b.reshape(1, HID) for b in (b1, b2, b3))
    bfc1r = bfc1.reshape(1, HID)
    bfc2r = bfc2.reshape(1, 1)

    degp = _sc_degree(dstp)
    y1, dinv = _tc_embed(xp, request, timestamp, b2d, degp,
                         wxp, wea, wet, bemb, W1)
    acc1 = _sc_scatter(y1, srcp, dstp)
    x1, y2 = _tc_layer(acc1, y1, dinv, b2d, b1r, W2)
    acc2 = _sc_scatter(y2, srcp, dstp)
    x2, y3 = _tc_layer(acc2, y2, dinv, b2d, b2r, W3)
    acc3 = _sc_scatter(y3, srcp, dstp)
    return _tc_head(acc3, y3, dinv, b2d, b3r, x1, x2,
                    Wfc1, bfc1r, Wfc2, bfc2r)


# speedup vs baseline: 1.1749x; 1.1749x over previous
import functools

import jax
import jax.numpy as jnp
from jax import lax
from jax.experimental import pallas as pl
from jax.experimental.pallas import tpu as pltpu
from jax.experimental.pallas import tpu_sc as plsc

N = 10000
E = 320000
B = 16
HID = 64
NPAD = 10240
EPAD = 327680
NTILES = 32
EPW = EPAD // NTILES
CH = 128
NCHUNK = EPW // CH
RPT = NPAD // 16
DEGW = 16
NC0 = 102
NC1 = 58
NCMAX = max(NC0, NC1)
BLK = 1280
NBLK = NPAD // BLK


@functools.cache
def _mesh():
    return plsc.VectorSubcoreMesh(
        core_axis_name="c", subcore_axis_name="s",
        num_cores=2, num_subcores=16)


def _fill(ref, rows, width, value):
    vec = jnp.full((16,), value, jnp.float32)

    def body(i, carry):
        for j in range(width // 16):
            ref[i, pl.ds(j * 16, 16)] = vec
        return carry

    lax.fori_loop(0, rows, body, 0)


def _sc_scatter_body(y_hbm, src_hbm, dst_hbm, out_hbm,
                     sidx, didx, rows0, rows1, zbuf, acc, sem0, sem1):
    cid = lax.axis_index("c")
    sid = lax.axis_index("s")
    _fill(zbuf, CH, HID, 0.0)
    row0 = sid * RPT
    for k in range(RPT // CH):
        pltpu.sync_copy(zbuf, acc.at[pl.ds(row0 + k * CH, CH)])
    @pl.when(cid == 0)
    def _stage0():
        c0 = sid * (NC0 + NC1)
        pltpu.sync_copy(src_hbm.at[pl.ds(c0, NC0)], sidx.at[pl.ds(0, NC0)])
        pltpu.sync_copy(dst_hbm.at[pl.ds(c0, NC0)], didx.at[pl.ds(0, NC0)])

    @pl.when(cid == 1)
    def _stage1():
        c0 = sid * (NC0 + NC1) + NC0
        pltpu.sync_copy(src_hbm.at[pl.ds(c0, NC1)], sidx.at[pl.ds(0, NC1)])
        pltpu.sync_copy(dst_hbm.at[pl.ds(c0, NC1)], didx.at[pl.ds(0, NC1)])

    plsc.subcore_barrier()

    def body(c, carry):
        pltpu.sync_copy(y_hbm.at[sidx.at[c]], rows0)
        pltpu.sync_copy(rows0, acc.at[didx.at[c]], add=True)
        return carry

    @pl.when(cid == 0)
    def _loop0():
        lax.fori_loop(0, NC0, body, 0)

    @pl.when(cid == 1)
    def _loop1():
        lax.fori_loop(0, NC1, body, 0)

    plsc.subcore_barrier()
    pltpu.sync_copy(acc.at[pl.ds(row0, RPT)],
                    out_hbm.at[cid, pl.ds(row0, RPT)])


def _sc_scatter(y, srcp, dstp):
    return pl.kernel(
        _sc_scatter_body,
        out_type=jax.ShapeDtypeStruct((2, NPAD, HID), jnp.float32),
        mesh=_mesh(),
        compiler_params=pltpu.CompilerParams(use_tc_tiling_on_sc=False),
        scratch_types=[
            pltpu.VMEM((NCMAX + 1, CH), jnp.int32),
            pltpu.VMEM((NCMAX, CH), jnp.int32),
            pltpu.VMEM((CH, HID), jnp.float32),
            pltpu.VMEM((CH, HID), jnp.float32),
            pltpu.VMEM((CH, HID), jnp.float32),
            pltpu.VMEM_SHARED((NPAD, HID), jnp.float32),
            pltpu.SemaphoreType.DMA,
            pltpu.SemaphoreType.DMA,
        ],
    )(y, srcp, dstp)


def _sc_degree_body(dst_hbm, out_hbm, didx, ones_v, zbuf, acc):
    cid = lax.axis_index("c")
    sid = lax.axis_index("s")
    wid = sid * 2 + cid
    _fill(zbuf, CH, DEGW, 0.0)
    _fill(ones_v, CH, DEGW, 1.0)
    row0 = sid * RPT
    for k in range(RPT // CH):
        pltpu.sync_copy(zbuf, acc.at[pl.ds(row0 + k * CH, CH)])
    pltpu.sync_copy(dst_hbm.at[pl.ds(wid * NCHUNK, NCHUNK)], didx)
    plsc.subcore_barrier()

    def body(c, carry):
        pltpu.sync_copy(ones_v, acc.at[didx.at[c]], add=True)
        return carry

    lax.fori_loop(0, NCHUNK, body, 0)
    plsc.subcore_barrier()
    pltpu.sync_copy(acc.at[pl.ds(row0, RPT)],
                    out_hbm.at[cid, pl.ds(row0, RPT)])


def _sc_degree(dstp):
    return pl.kernel(
        _sc_degree_body,
        out_type=jax.ShapeDtypeStruct((2, NPAD, DEGW), jnp.float32),
        mesh=_mesh(),
        compiler_params=pltpu.CompilerParams(use_tc_tiling_on_sc=False),
        scratch_types=[
            pltpu.VMEM((NCHUNK, CH), jnp.int32),
            pltpu.VMEM((CH, DEGW), jnp.float32),
            pltpu.VMEM((CH, DEGW), jnp.float32),
            pltpu.VMEM_SHARED((NPAD, DEGW), jnp.float32),
        ],
    )(dstp)


def _tc_embed_body(xp, req, ts, b2d, degp, wxp, wea, wet, bemb, w1,
                   y1_o, dinv_o):
    f32 = jnp.float32
    oh = (b2d[...] == lax.broadcasted_iota(jnp.int32, (BLK, B), 1)).astype(f32)
    z = req[:, 0:1] * wea[...] + jnp.dot(ts[...], wet[...],
                                         preferred_element_type=f32,
                  precision=lax.Precision.HIGHEST)
    h0 = (jnp.dot(xp[...], wxp[...], preferred_element_type=f32,
                  precision=lax.Precision.HIGHEST)
          + jnp.dot(oh, z, preferred_element_type=f32,
                  precision=lax.Precision.HIGHEST) + bemb[...])
    deg = degp[0, :, 0:1] + degp[1, :, 0:1] + 1.0
    dinv = 1.0 / jnp.sqrt(deg)
    dinv_o[...] = dinv
    y1_o[...] = dinv * jnp.dot(h0, w1[...], preferred_element_type=f32,
                  precision=lax.Precision.HIGHEST)


def _tc_embed(xp, req, ts, b2d, degp, wxp, wea, wet, bemb, w1):
    return pl.pallas_call(
        _tc_embed_body,
        grid=(NBLK,),
        in_specs=[
            pl.BlockSpec((BLK, 128), lambda i: (i, 0)),
            pl.BlockSpec((B, 4), lambda i: (0, 0)),
            pl.BlockSpec((B, 4), lambda i: (0, 0)),
            pl.BlockSpec((BLK, 1), lambda i: (i, 0)),
            pl.BlockSpec((2, BLK, DEGW), lambda i: (0, i, 0)),
            pl.BlockSpec((128, HID), lambda i: (0, 0)),
            pl.BlockSpec((1, HID), lambda i: (0, 0)),
            pl.BlockSpec((4, HID), lambda i: (0, 0)),
            pl.BlockSpec((1, HID), lambda i: (0, 0)),
            pl.BlockSpec((HID, HID), lambda i: (0, 0)),
        ],
        out_specs=[
            pl.BlockSpec((BLK, HID), lambda i: (i, 0)),
            pl.BlockSpec((BLK, 1), lambda i: (i, 0)),
        ],
        out_shape=[
            jax.ShapeDtypeStruct((NPAD, HID), jnp.float32),
            jax.ShapeDtypeStruct((NPAD, 1), jnp.float32),
        ],
    )(xp, req, ts, b2d, degp, wxp, wea, wet, bemb, w1)


def _pool_update(h, b2d, s_acc, c_acc, m_acc):
    f32 = jnp.float32
    pid = pl.program_id(0)
    oh = (b2d == lax.broadcasted_iota(jnp.int32, (BLK, B), 1)).astype(f32)
    dims = (((0,), (0,)), ((), ()))
    s = lax.dot_general(oh, h, dims, preferred_element_type=f32,
                  precision=lax.Precision.HIGHEST)
    c = lax.dot_general(oh, jnp.ones((BLK, 1), f32), dims,
                        preferred_element_type=f32,
                  precision=lax.Precision.HIGHEST)
    neg = jnp.float32(-jnp.inf)
    mxs = [jnp.max(jnp.where(b2d == g, h, neg), axis=0, keepdims=True)
           for g in range(B)]
    m = jnp.concatenate(mxs, axis=0)

    @pl.when(pid == 0)
    def _init():
        s_acc[...] = s
        c_acc[...] = c
        m_acc[...] = m

    @pl.when(pid > 0)
    def _upd():
        s_acc[...] += s
        c_acc[...] += c
        m_acc[...] = jnp.maximum(m_acc[...], m)


def _layer_h(accp, yprev, dinv, bl):
    return jnp.maximum(
        dinv[...] * (accp[0] + accp[1] + yprev[...]) + bl[...], 0.0)


def _tc_layer_body(accp, yprev, dinv, b2d, bl, wnext,
                   xl_o, ynext_o, s_acc, c_acc, m_acc):
    f32 = jnp.float32
    h = _layer_h(accp, yprev, dinv, bl)
    _pool_update(h, b2d[...], s_acc, c_acc, m_acc)
    ynext_o[...] = dinv[...] * jnp.dot(h, wnext[...],
                                       preferred_element_type=f32,
                  precision=lax.Precision.HIGHEST)

    @pl.when(pl.program_id(0) == NBLK - 1)
    def _fin():
        mean = s_acc[...] / jnp.maximum(c_acc[...], 1.0)
        xl_o[...] = jnp.concatenate([mean, m_acc[...]], axis=1)


def _tc_layer(accp, yprev, dinv, b2d, bl, wnext):
    return pl.pallas_call(
        _tc_layer_body,
        grid=(NBLK,),
        in_specs=[
            pl.BlockSpec((2, BLK, HID), lambda i: (0, i, 0)),
            pl.BlockSpec((BLK, HID), lambda i: (i, 0)),
            pl.BlockSpec((BLK, 1), lambda i: (i, 0)),
            pl.BlockSpec((BLK, 1), lambda i: (i, 0)),
            pl.BlockSpec((1, HID), lambda i: (0, 0)),
            pl.BlockSpec((HID, HID), lambda i: (0, 0)),
        ],
        out_specs=[
            pl.BlockSpec((B, 2 * HID), lambda i: (0, 0)),
            pl.BlockSpec((BLK, HID), lambda i: (i, 0)),
        ],
        out_shape=[
            jax.ShapeDtypeStruct((B, 2 * HID), jnp.float32),
            jax.ShapeDtypeStruct((NPAD, HID), jnp.float32),
        ],
        scratch_shapes=[
            pltpu.VMEM((B, HID), jnp.float32),
            pltpu.VMEM((B, 1), jnp.float32),
            pltpu.VMEM((B, HID), jnp.float32),
        ],
    )(accp, yprev, dinv, b2d, bl, wnext)


def _tc_head_body(accp, yprev, dinv, b2d, bl, x1, x2, wfc1, bfc1, wfc2, bfc2,
                  out_o, s_acc, c_acc, m_acc):
    f32 = jnp.float32
    h = _layer_h(accp, yprev, dinv, bl)
    _pool_update(h, b2d[...], s_acc, c_acc, m_acc)

    @pl.when(pl.program_id(0) == NBLK - 1)
    def _fin():
        mean = s_acc[...] / jnp.maximum(c_acc[...], 1.0)
        x3 = jnp.concatenate([mean, m_acc[...]], axis=1)
        g = x1[...] + x2[...] + x3
        t = jnp.maximum(jnp.dot(g, wfc1[...], preferred_element_type=f32,
                  precision=lax.Precision.HIGHEST)
                        + bfc1[...], 0.0)
        out_o[...] = jnp.dot(t, wfc2[...], preferred_element_type=f32,
                  precision=lax.Precision.HIGHEST) \
            + bfc2[...]


def _tc_head(accp, yprev, dinv, b2d, bl, x1, x2, wfc1, bfc1, wfc2, bfc2):
    return pl.pallas_call(
        _tc_head_body,
        grid=(NBLK,),
        in_specs=[
            pl.BlockSpec((2, BLK, HID), lambda i: (0, i, 0)),
            pl.BlockSpec((BLK, HID), lambda i: (i, 0)),
            pl.BlockSpec((BLK, 1), lambda i: (i, 0)),
            pl.BlockSpec((BLK, 1), lambda i: (i, 0)),
            pl.BlockSpec((1, HID), lambda i: (0, 0)),
            pl.BlockSpec((B, 2 * HID), lambda i: (0, 0)),
            pl.BlockSpec((B, 2 * HID), lambda i: (0, 0)),
            pl.BlockSpec((2 * HID, HID), lambda i: (0, 0)),
            pl.BlockSpec((1, HID), lambda i: (0, 0)),
            pl.BlockSpec((HID, 1), lambda i: (0, 0)),
            pl.BlockSpec((1, 1), lambda i: (0, 0)),
        ],
        out_specs=pl.BlockSpec((B, 1), lambda i: (0, 0)),
        out_shape=jax.ShapeDtypeStruct((B, 1), jnp.float32),
        scratch_shapes=[
            pltpu.VMEM((B, HID), jnp.float32),
            pltpu.VMEM((B, 1), jnp.float32),
            pltpu.VMEM((B, HID), jnp.float32),
        ],
    )(accp, yprev, dinv, b2d, bl, x1, x2, wfc1, bfc1, wfc2, bfc2)


def kernel(x, edge_index, batch, request, timestamp, W_embed, b_embed,
           W1, b1, W2, b2, W3, b3, Wfc1, bfc1, Wfc2, bfc2):
    npad = NPAD - N
    xp = jnp.pad(x, ((0, npad), (0, 5)))
    wxp = jnp.pad(W_embed[:123], ((0, 5), (0, 0)))
    wea = W_embed[123:124]
    wet = W_embed[124:128]
    b2d = jnp.pad(batch[:, None], ((0, npad), (0, 0)),
                  constant_values=B)
    pad = EPAD - E
    srcp = jnp.concatenate(
        [edge_index[0], jnp.zeros((pad,), jnp.int32)]).reshape(
            EPAD // CH, CH)
    dstp = jnp.concatenate(
        [edge_index[1], jnp.full((pad,), N, jnp.int32)]).reshape(
            EPAD // CH, CH)
    bemb = b_embed.reshape(1, HID)
    b1r, b2r, b3r = (b.reshape(1, HID) for b in (b1, b2, b3))
    bfc1r = bfc1.reshape(1, HID)
    bfc2r = bfc2.reshape(1, 1)

    degp = _sc_degree(dstp)
    y1, dinv = _tc_embed(xp, request, timestamp, b2d, degp,
                         wxp, wea, wet, bemb, W1)
    acc1 = _sc_scatter(y1, srcp, dstp)
    x1, y2 = _tc_layer(acc1, y1, dinv, b2d, b1r, W2)
    acc2 = _sc_scatter(y2, srcp, dstp)
    x2, y3 = _tc_layer(acc2, y2, dinv, b2d, b2r, W3)
    acc3 = _sc_scatter(y3, srcp, dstp)
    return _tc_head(acc3, y3, dinv, b2d, b3r, x1, x2,
                    Wfc1, bfc1r, Wfc2, bfc2r)


# speedup vs baseline: 1.3144x; 1.1187x over previous
import functools

import jax
import jax.numpy as jnp
from jax import lax
from jax.experimental import pallas as pl
from jax.experimental.pallas import tpu as pltpu
from jax.experimental.pallas import tpu_sc as plsc

N = 10000
E = 320000
B = 16
HID = 64
NPAD = 10240
EPAD = 327680
NTILES = 32
EPW = EPAD // NTILES
CH = 128
NCHUNK = EPW // CH
RPT = NPAD // 16
DEGW = 16
NC0 = 113
NC1 = 47
NCMAX = max(NC0, NC1)
BLK = 1280
NBLK = NPAD // BLK


@functools.cache
def _mesh():
    return plsc.VectorSubcoreMesh(
        core_axis_name="c", subcore_axis_name="s",
        num_cores=2, num_subcores=16)


def _fill(ref, rows, width, value):
    vec = jnp.full((16,), value, jnp.float32)

    def body(i, carry):
        for j in range(width // 16):
            ref[i, pl.ds(j * 16, 16)] = vec
        return carry

    lax.fori_loop(0, rows, body, 0)


def _sc_scatter_body(y_hbm, src_hbm, dst_hbm, out_hbm,
                     sidx, didx, rows0, rows1, zbuf, acc, sem0, sem1):
    cid = lax.axis_index("c")
    sid = lax.axis_index("s")
    _fill(zbuf, CH, HID, 0.0)
    row0 = sid * RPT
    for k in range(RPT // CH):
        pltpu.sync_copy(zbuf, acc.at[pl.ds(row0 + k * CH, CH)])
    @pl.when(cid == 0)
    def _stage0():
        c0 = sid * (NC0 + NC1)
        pltpu.sync_copy(src_hbm.at[pl.ds(c0, NC0)], sidx.at[pl.ds(0, NC0)])
        pltpu.sync_copy(dst_hbm.at[pl.ds(c0, NC0)], didx.at[pl.ds(0, NC0)])

    @pl.when(cid == 1)
    def _stage1():
        c0 = sid * (NC0 + NC1) + NC0
        pltpu.sync_copy(src_hbm.at[pl.ds(c0, NC1)], sidx.at[pl.ds(0, NC1)])
        pltpu.sync_copy(dst_hbm.at[pl.ds(c0, NC1)], didx.at[pl.ds(0, NC1)])

    plsc.subcore_barrier()

    def body(c, carry):
        pltpu.sync_copy(y_hbm.at[sidx.at[c]], rows0)
        pltpu.sync_copy(rows0, acc.at[didx.at[c]], add=True)
        return carry

    @pl.when(cid == 0)
    def _loop0():
        lax.fori_loop(0, NC0, body, 0)

    @pl.when(cid == 1)
    def _loop1():
        lax.fori_loop(0, NC1, body, 0)

    plsc.subcore_barrier()
    pltpu.sync_copy(acc.at[pl.ds(row0, RPT)],
                    out_hbm.at[cid, pl.ds(row0, RPT)])


def _sc_scatter(y, srcp, dstp):
    return pl.kernel(
        _sc_scatter_body,
        out_type=jax.ShapeDtypeStruct((2, NPAD, HID), jnp.float32),
        mesh=_mesh(),
        compiler_params=pltpu.CompilerParams(use_tc_tiling_on_sc=False),
        scratch_types=[
            pltpu.VMEM((NCMAX + 1, CH), jnp.int32),
            pltpu.VMEM((NCMAX, CH), jnp.int32),
            pltpu.VMEM((CH, HID), jnp.float32),
            pltpu.VMEM((CH, HID), jnp.float32),
            pltpu.VMEM((CH, HID), jnp.float32),
            pltpu.VMEM_SHARED((NPAD, HID), jnp.float32),
            pltpu.SemaphoreType.DMA,
            pltpu.SemaphoreType.DMA,
        ],
    )(y, srcp, dstp)


def _sc_degree_body(dst_hbm, out_hbm, didx, ones_v, zbuf, acc):
    cid = lax.axis_index("c")
    sid = lax.axis_index("s")
    wid = sid * 2 + cid
    _fill(zbuf, CH, DEGW, 0.0)
    _fill(ones_v, CH, DEGW, 1.0)
    row0 = sid * RPT
    for k in range(RPT // CH):
        pltpu.sync_copy(zbuf, acc.at[pl.ds(row0 + k * CH, CH)])
    pltpu.sync_copy(dst_hbm.at[pl.ds(wid * NCHUNK, NCHUNK)], didx)
    plsc.subcore_barrier()

    def body(c, carry):
        pltpu.sync_copy(ones_v, acc.at[didx.at[c]], add=True)
        return carry

    lax.fori_loop(0, NCHUNK, body, 0)
    plsc.subcore_barrier()
    pltpu.sync_copy(acc.at[pl.ds(row0, RPT)],
                    out_hbm.at[cid, pl.ds(row0, RPT)])


def _sc_degree(dstp):
    return pl.kernel(
        _sc_degree_body,
        out_type=jax.ShapeDtypeStruct((2, NPAD, DEGW), jnp.float32),
        mesh=_mesh(),
        compiler_params=pltpu.CompilerParams(use_tc_tiling_on_sc=False),
        scratch_types=[
            pltpu.VMEM((NCHUNK, CH), jnp.int32),
            pltpu.VMEM((CH, DEGW), jnp.float32),
            pltpu.VMEM((CH, DEGW), jnp.float32),
            pltpu.VMEM_SHARED((NPAD, DEGW), jnp.float32),
        ],
    )(dstp)


def _tc_embed_body(xp, req, ts, b2d, degp, wxp, wea, wet, bemb, w1,
                   y1_o, dinv_o):
    f32 = jnp.float32
    oh = (b2d[...] == lax.broadcasted_iota(jnp.int32, (BLK, B), 1)).astype(f32)
    z = req[:, 0:1] * wea[...] + jnp.dot(ts[...], wet[...],
                                         preferred_element_type=f32,
                  precision=lax.Precision.HIGHEST)
    h0 = (jnp.dot(xp[...], wxp[...], preferred_element_type=f32,
                  precision=lax.Precision.HIGHEST)
          + jnp.dot(oh, z, preferred_element_type=f32,
                  precision=lax.Precision.HIGHEST) + bemb[...])
    deg = degp[0, :, 0:1] + degp[1, :, 0:1] + 1.0
    dinv = 1.0 / jnp.sqrt(deg)
    dinv_o[...] = dinv
    y1_o[...] = dinv * jnp.dot(h0, w1[...], preferred_element_type=f32,
                  precision=lax.Precision.HIGHEST)


def _tc_embed(xp, req, ts, b2d, degp, wxp, wea, wet, bemb, w1):
    return pl.pallas_call(
        _tc_embed_body,
        grid=(NBLK,),
        in_specs=[
            pl.BlockSpec((BLK, 128), lambda i: (i, 0)),
            pl.BlockSpec((B, 4), lambda i: (0, 0)),
            pl.BlockSpec((B, 4), lambda i: (0, 0)),
            pl.BlockSpec((BLK, 1), lambda i: (i, 0)),
            pl.BlockSpec((2, BLK, DEGW), lambda i: (0, i, 0)),
            pl.BlockSpec((128, HID), lambda i: (0, 0)),
            pl.BlockSpec((1, HID), lambda i: (0, 0)),
            pl.BlockSpec((4, HID), lambda i: (0, 0)),
            pl.BlockSpec((1, HID), lambda i: (0, 0)),
            pl.BlockSpec((HID, HID), lambda i: (0, 0)),
        ],
        out_specs=[
            pl.BlockSpec((BLK, HID), lambda i: (i, 0)),
            pl.BlockSpec((BLK, 1), lambda i: (i, 0)),
        ],
        out_shape=[
            jax.ShapeDtypeStruct((NPAD, HID), jnp.float32),
            jax.ShapeDtypeStruct((NPAD, 1), jnp.float32),
        ],
    )(xp, req, ts, b2d, degp, wxp, wea, wet, bemb, w1)


def _pool_update(h, b2d, s_acc, c_acc, m_acc):
    f32 = jnp.float32
    pid = pl.program_id(0)
    oh = (b2d == lax.broadcasted_iota(jnp.int32, (BLK, B), 1)).astype(f32)
    dims = (((0,), (0,)), ((), ()))
    s = lax.dot_general(oh, h, dims, preferred_element_type=f32,
                  precision=lax.Precision.HIGHEST)
    c = lax.dot_general(oh, jnp.ones((BLK, 1), f32), dims,
                        preferred_element_type=f32,
                  precision=lax.Precision.HIGHEST)
    neg = jnp.float32(-jnp.inf)
    mxs = [jnp.max(jnp.where(b2d == g, h, neg), axis=0, keepdims=True)
           for g in range(B)]
    m = jnp.concatenate(mxs, axis=0)

    @pl.when(pid == 0)
    def _init():
        s_acc[...] = s
        c_acc[...] = c
        m_acc[...] = m

    @pl.when(pid > 0)
    def _upd():
        s_acc[...] += s
        c_acc[...] += c
        m_acc[...] = jnp.maximum(m_acc[...], m)


def _layer_h(accp, yprev, dinv, bl):
    return jnp.maximum(
        dinv[...] * (accp[0] + accp[1] + yprev[...]) + bl[...], 0.0)


def _tc_layer_body(accp, yprev, dinv, b2d, bl, wnext,
                   xl_o, ynext_o, s_acc, c_acc, m_acc):
    f32 = jnp.float32
    h = _layer_h(accp, yprev, dinv, bl)
    _pool_update(h, b2d[...], s_acc, c_acc, m_acc)
    ynext_o[...] = dinv[...] * jnp.dot(h, wnext[...],
                                       preferred_element_type=f32,
                  precision=lax.Precision.HIGHEST)

    @pl.when(pl.program_id(0) == NBLK - 1)
    def _fin():
        mean = s_acc[...] / jnp.maximum(c_acc[...], 1.0)
        xl_o[...] = jnp.concatenate([mean, m_acc[...]], axis=1)


def _tc_layer(accp, yprev, dinv, b2d, bl, wnext):
    return pl.pallas_call(
        _tc_layer_body,
        grid=(NBLK,),
        in_specs=[
            pl.BlockSpec((2, BLK, HID), lambda i: (0, i, 0)),
            pl.BlockSpec((BLK, HID), lambda i: (i, 0)),
            pl.BlockSpec((BLK, 1), lambda i: (i, 0)),
            pl.BlockSpec((BLK, 1), lambda i: (i, 0)),
            pl.BlockSpec((1, HID), lambda i: (0, 0)),
            pl.BlockSpec((HID, HID), lambda i: (0, 0)),
        ],
        out_specs=[
            pl.BlockSpec((B, 2 * HID), lambda i: (0, 0)),
            pl.BlockSpec((BLK, HID), lambda i: (i, 0)),
        ],
        out_shape=[
            jax.ShapeDtypeStruct((B, 2 * HID), jnp.float32),
            jax.ShapeDtypeStruct((NPAD, HID), jnp.float32),
        ],
        scratch_shapes=[
            pltpu.VMEM((B, HID), jnp.float32),
            pltpu.VMEM((B, 1), jnp.float32),
            pltpu.VMEM((B, HID), jnp.float32),
        ],
    )(accp, yprev, dinv, b2d, bl, wnext)


def _tc_head_body(accp, yprev, dinv, b2d, bl, x1, x2, wfc1, bfc1, wfc2, bfc2,
                  out_o, s_acc, c_acc, m_acc):
    f32 = jnp.float32
    h = _layer_h(accp, yprev, dinv, bl)
    _pool_update(h, b2d[...], s_acc, c_acc, m_acc)

    @pl.when(pl.program_id(0) == NBLK - 1)
    def _fin():
        mean = s_acc[...] / jnp.maximum(c_acc[...], 1.0)
        x3 = jnp.concatenate([mean, m_acc[...]], axis=1)
        g = x1[...] + x2[...] + x3
        t = jnp.maximum(jnp.dot(g, wfc1[...], preferred_element_type=f32,
                  precision=lax.Precision.HIGHEST)
                        + bfc1[...], 0.0)
        out_o[...] = jnp.dot(t, wfc2[...], preferred_element_type=f32,
                  precision=lax.Precision.HIGHEST) \
            + bfc2[...]


def _tc_head(accp, yprev, dinv, b2d, bl, x1, x2, wfc1, bfc1, wfc2, bfc2):
    return pl.pallas_call(
        _tc_head_body,
        grid=(NBLK,),
        in_specs=[
            pl.BlockSpec((2, BLK, HID), lambda i: (0, i, 0)),
            pl.BlockSpec((BLK, HID), lambda i: (i, 0)),
            pl.BlockSpec((BLK, 1), lambda i: (i, 0)),
            pl.BlockSpec((BLK, 1), lambda i: (i, 0)),
            pl.BlockSpec((1, HID), lambda i: (0, 0)),
            pl.BlockSpec((B, 2 * HID), lambda i: (0, 0)),
            pl.BlockSpec((B, 2 * HID), lambda i: (0, 0)),
            pl.BlockSpec((2 * HID, HID), lambda i: (0, 0)),
            pl.BlockSpec((1, HID), lambda i: (0, 0)),
            pl.BlockSpec((HID, 1), lambda i: (0, 0)),
            pl.BlockSpec((1, 1), lambda i: (0, 0)),
        ],
        out_specs=pl.BlockSpec((B, 1), lambda i: (0, 0)),
        out_shape=jax.ShapeDtypeStruct((B, 1), jnp.float32),
        scratch_shapes=[
            pltpu.VMEM((B, HID), jnp.float32),
            pltpu.VMEM((B, 1), jnp.float32),
            pltpu.VMEM((B, HID), jnp.float32),
        ],
    )(accp, yprev, dinv, b2d, bl, x1, x2, wfc1, bfc1, wfc2, bfc2)


def kernel(x, edge_index, batch, request, timestamp, W_embed, b_embed,
           W1, b1, W2, b2, W3, b3, Wfc1, bfc1, Wfc2, bfc2):
    npad = NPAD - N
    xp = jnp.pad(x, ((0, npad), (0, 5)))
    wxp = jnp.pad(W_embed[:123], ((0, 5), (0, 0)))
    wea = W_embed[123:124]
    wet = W_embed[124:128]
    b2d = jnp.pad(batch[:, None], ((0, npad), (0, 0)),
                  constant_values=B)
    pad = EPAD - E
    srcp = jnp.concatenate(
        [edge_index[0], jnp.zeros((pad,), jnp.int32)]).reshape(
            EPAD // CH, CH)
    dstp = jnp.concatenate(
        [edge_index[1], jnp.full((pad,), N, jnp.int32)]).reshape(
            EPAD // CH, CH)
    bemb = b_embed.reshape(1, HID)
    b1r, b2r, b3r = (b.reshape(1, HID) for b in (b1, b2, b3))
    bfc1r = bfc1.reshape(1, HID)
    bfc2r = bfc2.reshape(1, 1)

    degp = _sc_degree(dstp)
    y1, dinv = _tc_embed(xp, request, timestamp, b2d, degp,
                         wxp, wea, wet, bemb, W1)
    acc1 = _sc_scatter(y1, srcp, dstp)
    x1, y2 = _tc_layer(acc1, y1, dinv, b2d, b1r, W2)
    acc2 = _sc_scatter(y2, srcp, dstp)
    x2, y3 = _tc_layer(acc2, y2, dinv, b2d, b2r, W3)
    acc3 = _sc_scatter(y3, srcp, dstp)
    return _tc_head(acc3, y3, dinv, b2d, b3r, x1, x2,
                    Wfc1, bfc1r, Wfc2, bfc2r)
